# baseline (speedup 1.0000x reference)
import functools

import jax
import jax.numpy as jnp
from jax import lax
from jax.experimental import pallas as pl
from jax.experimental.pallas import tpu as pltpu
from jax.experimental.pallas import tpu_sc as plsc

N = 10000
E = 320000
B = 64
H = 128
L = 4

NPAD = 10240
NCORE = 2
NSUB = 16
CHUNK = 128
CPT = 80
PER_TILE = CHUNK * CPT
EPAD = NCORE * NSUB * PER_TILE
TW = 256
SW = 128
EBLK = 1024
NBLK = 1024
NPB = NPAD // NBLK

_f32 = jnp.float32
_bf16 = jnp.bfloat16


def _silu(t):
    return t * jax.nn.sigmoid(t)


def _ln(t, g, b):
    mu = jnp.mean(t, -1, keepdims=True)
    d = t - mu
    var = jnp.mean(d * d, -1, keepdims=True)
    return d / jnp.sqrt(var + 1e-5) * g + b


def _wspec(shape):
    nd = len(shape)
    return pl.BlockSpec(shape, lambda i: (0,) * nd)


def _bspec(shape):
    return pl.BlockSpec(shape, lambda i: (i,) + (0,) * (len(shape) - 1))


def _tables(h, x16, wa, wb):
    z = jnp.zeros((h.shape[0], TW - 144), _f32)
    a = jnp.concatenate(
        [jnp.dot(h, wa[...], preferred_element_type=_f32), x16, z], axis=1)
    b = jnp.concatenate(
        [jnp.dot(h, wb[...], preferred_element_type=_f32), x16, z], axis=1)
    return a, b


def _enc_body(hin, x16, new, neb, neg, nebeta, wa, wb, h_out, a_out, b_out):
    hp = _silu(jnp.dot(hin[...], new[...], preferred_element_type=_f32) + neb[...])
    h = _ln(hp, neg[...], nebeta[...])
    h_out[...] = h
    a_out[...], b_out[...] = _tables(h, x16[...], wa, wb)


def _edge_enc_body(eap, eew, eeb, out):
    out[...] = _silu(jnp.dot(eap[...], eew[...], preferred_element_type=_f32)
                     + eeb[...])


def _edge_body(has_coord, av, bv, ea, wd, wc, eb1, ew2, eb2, cw1, cb1, cw2,
               m_out, t_out=None):
    a = av[...]
    b = bv[...]
    hsum = a[:, 0:128] + b[:, 0:128]
    xd16 = a[:, 128:144] - b[:, 128:144]
    dist_sq = jnp.sum(xd16 * xd16, axis=1, keepdims=True)
    e1 = (hsum
          + jnp.dot(ea[...], wd[...], preferred_element_type=_f32)
          + dist_sq * wc[...] + eb1[...])
    m = _silu(jnp.dot(_silu(e1), ew2[...], preferred_element_type=_f32) + eb2[...])
    m_out[...] = m
    if has_coord:
        lane = lax.broadcasted_iota(jnp.int32, (EBLK, 16), 1)
        t = _silu(jnp.dot(m, cw1[...], preferred_element_type=_f32) + cb1[...])
        c = jnp.tanh(jnp.dot(t, cw2[...], preferred_element_type=_f32))
        dist = jnp.sqrt(dist_sq + 1e-8)
        tail = xd16 * (c / dist)
        tail = jnp.where(lane == 3, 1.0, tail)
        t_out[...] = jnp.concatenate(
            [tail, jnp.zeros((EBLK, SW - 16), _f32)], axis=1)


def _node_body(has_coord, *refs):
    if has_coord:
        (h_in, x16, p0, p1, t0, t1, nw1h, nw1m, nb1, nw2, nb2, lng, lnb,
         wa, wb, h_out, x_out, a_out, b_out) = refs
    else:
        h_in, p0, p1, nw1h, nw1m, nb1, nw2, nb2, lng, lnb, h_out = refs
    h = h_in[...]
    m_i = p0[:, 0:128] + p1[:, 0:128]
    hu = _silu(jnp.dot(h, nw1h[...], preferred_element_type=_f32)
               + jnp.dot(m_i, nw1m[...], preferred_element_type=_f32)
               + nb1[...])
    hu = jnp.dot(hu, nw2[...], preferred_element_type=_f32) + nb2[...]
    hn = _ln(h + hu, lng[...], lnb[...])
    h_out[...] = hn
    if has_coord:
        tail = t0[:, 0:16] + t1[:, 0:16]
        deg = jnp.maximum(tail[:, 3:4], 1.0)
        lane = lax.broadcasted_iota(jnp.int32, (NBLK, 16), 1)
        xn = x16[...] + jnp.where(lane < 3, tail, 0.0) / deg
        x_out[...] = xn
        a_out[...], b_out[...] = _tables(hn, xn, wa, wb)


def _pool_body(h_ref, bp_ref, pw1, pb1, pw2, pb2, cw1, cb1, cw2, cb2, cw3, cb3,
               out_ref):
    h = h_ref[...]
    bp = bp_ref[...]
    s = jnp.dot(jnp.tanh(jnp.dot(h, pw1[...], preferred_element_type=_f32)
                         + pb1[...]),
                pw2[...], preferred_element_type=_f32) + pb2[...]
    cols = lax.broadcasted_iota(jnp.int32, (NPAD, B), 1)
    m = bp == cols
    mf = m.astype(_f32)
    dn = (((0,), (0,)), ((), ()))
    smax = jnp.max(jnp.where(m, s, -1e30), axis=0, keepdims=True)
    sg = jnp.sum(jnp.where(m, smax, 0.0), axis=1, keepdims=True)
    sexp = jnp.exp(s - sg)
    ssum = lax.dot_general(sexp, mf, dn, preferred_element_type=_f32)
    sden = jnp.sum(jnp.where(m, ssum, 0.0), axis=1, keepdims=True)
    w = jnp.where(bp >= 0, sexp / (sden + 1e-16), 0.0)
    g = lax.dot_general(mf, h * w, dn, preferred_element_type=_f32)
    inv = 0.9999950000374996
    z = _silu(jnp.dot(g, cw1[...], preferred_element_type=_f32) + cb1[...]) * inv
    z = _silu(jnp.dot(z, cw2[...], preferred_element_type=_f32) + cb2[...]) * inv
    out_ref[...] = jnp.dot(z, cw3[...], preferred_element_type=_f32) + cb3[...]


def _gather_body(tab, idx_hbm, out, idx2d, buf0, buf1, sg0, sg1, sw0, sw1):
    wid = lax.axis_index("c") * NSUB + lax.axis_index("s")
    cbase = wid * CPT
    ebase = cbase * CHUNK
    pltpu.sync_copy(idx_hbm.at[pl.ds(cbase, CPT)], idx2d)
    bufs = (buf0, buf1)
    sgs = (sg0, sg1)
    sws = (sw0, sw1)

    pltpu.async_copy(tab.at[idx2d.at[0]], buf0, sg0)

    @pl.loop(0, CPT, step=2)
    def _pipe(t0):
        for bpar in range(2):
            t = t0 + bpar
            buf, sg, sw = bufs[bpar], sgs[bpar], sws[bpar]
            obuf, osg, osw = bufs[1 - bpar], sgs[1 - bpar], sws[1 - bpar]

            @pl.when(t >= 1)
            def _():
                pltpu.make_async_copy(
                    obuf, out.at[pl.ds(ebase + (t - 1) * CHUNK, CHUNK)],
                    osw).wait()

            @pl.when(t + 1 < CPT)
            def _():
                pltpu.async_copy(tab.at[idx2d.at[t + 1]], obuf, osg)

            pltpu.make_async_copy(tab.at[idx2d.at[t]], buf, sg).wait()
            pltpu.async_copy(buf, out.at[pl.ds(ebase + t * CHUNK, CHUNK)], sw)

    pltpu.make_async_copy(
        buf1, out.at[pl.ds(ebase + (CPT - 1) * CHUNK, CHUNK)], sw1).wait()


_gather = pl.kernel(
    _gather_body,
    out_type=jax.ShapeDtypeStruct((EPAD, TW), _f32),
    mesh=plsc.VectorSubcoreMesh(core_axis_name="c", subcore_axis_name="s"),
    scratch_types=[
        pltpu.VMEM((CPT, CHUNK), jnp.int32),
        pltpu.VMEM((CHUNK, TW), _f32),
        pltpu.VMEM((CHUNK, TW), _f32),
        pltpu.SemaphoreType.DMA,
        pltpu.SemaphoreType.DMA,
        pltpu.SemaphoreType.DMA,
        pltpu.SemaphoreType.DMA,
    ],
)


def _scatter_body(idx_hbm, mv_hbm, zero_hbm, out_hbm, idx2d, buf0, buf1,
                  acc_sh, sv0, sv1, ss0, ss1):
    cid = lax.axis_index("c")
    sid = lax.axis_index("s")
    wid = cid * NSUB + sid
    rows = NPAD // NSUB
    rbase = sid * rows
    cbase = wid * CPT
    ebase = cbase * CHUNK
    pltpu.sync_copy(idx_hbm.at[pl.ds(cbase, CPT)], idx2d)
    pltpu.sync_copy(zero_hbm.at[pl.ds(rbase, rows)],
                    acc_sh.at[pl.ds(rbase, rows)])
    plsc.subcore_barrier()

    bufs = (buf0, buf1)
    svs = (sv0, sv1)
    sss = (ss0, ss1)

    pltpu.async_copy(mv_hbm.at[pl.ds(ebase, CHUNK)], buf0, sv0)

    @pl.loop(0, CPT, step=2)
    def _pipe(t0):
        for bpar in range(2):
            t = t0 + bpar
            buf, sv, ss = bufs[bpar], svs[bpar], sss[bpar]
            obuf, osv, oss = bufs[1 - bpar], svs[1 - bpar], sss[1 - bpar]

            @pl.when(t + 1 < CPT)
            def _():
                @pl.when(t >= 1)
                def _():
                    pltpu.make_async_copy(
                        obuf, acc_sh.at[idx2d.at[t - 1]], oss).wait()
                pltpu.async_copy(
                    mv_hbm.at[pl.ds(ebase + (t + 1) * CHUNK, CHUNK)],
                    obuf, osv)

            pltpu.make_async_copy(
                mv_hbm.at[pl.ds(ebase + t * CHUNK, CHUNK)], buf, sv).wait()
            pltpu.async_copy(buf, acc_sh.at[idx2d.at[t]], ss, add=True)

    pltpu.make_async_copy(buf0, acc_sh.at[idx2d.at[CPT - 2]], ss0).wait()
    pltpu.make_async_copy(buf1, acc_sh.at[idx2d.at[CPT - 1]], ss1).wait()
    plsc.subcore_barrier()
    obase = cid * NPAD + rbase
    pltpu.sync_copy(acc_sh.at[pl.ds(rbase, rows)],
                    out_hbm.at[pl.ds(obase, rows)])


_scatter = pl.kernel(
    _scatter_body,
    out_type=jax.ShapeDtypeStruct((NCORE * NPAD, SW), _f32),
    mesh=plsc.VectorSubcoreMesh(core_axis_name="c", subcore_axis_name="s"),
    scratch_types=[
        pltpu.VMEM((CPT, CHUNK), jnp.int32),
        pltpu.VMEM((CHUNK, SW), _f32),
        pltpu.VMEM((CHUNK, SW), _f32),
        pltpu.VMEM_SHARED((NPAD, SW), _f32),
        pltpu.SemaphoreType.DMA,
        pltpu.SemaphoreType.DMA,
        pltpu.SemaphoreType.DMA,
        pltpu.SemaphoreType.DMA,
    ],
)


def kernel(h, x, edge_index, edge_attr, batch, params):
    p = params
    r2 = lambda t: t.reshape(1, -1)

    hp = jnp.pad(h, ((0, NPAD - N), (0, 64 - 58)))
    x16 = jnp.pad(x, ((0, NPAD - N), (0, 13)))
    eap = jnp.pad(edge_attr, ((0, EPAD - E), (0, 4)))
    rowp = jnp.pad(edge_index[0], (0, EPAD - E),
                   constant_values=NPAD - 1).reshape(EPAD // CHUNK, CHUNK)
    colp = jnp.pad(edge_index[1], (0, EPAD - E),
                   constant_values=NPAD - 1).reshape(EPAD // CHUNK, CHUNK)
    bp = jnp.pad(batch, (0, NPAD - N), constant_values=-1).reshape(NPAD, 1)
    zeros_acc = jnp.zeros((NPAD, SW), _f32)
    new_p = jnp.pad(p['ne_w'], ((0, 6), (0, 0)))
    eew_p = jnp.pad(p['ee_w'], ((0, 4), (0, 0)))

    lw = p['layers']
    wa0 = lw[0]['ew1'][0:128]
    wb0 = lw[0]['ew1'][128:256]

    grid_n = (NPB,)
    grid_e = (EPAD // EBLK,)

    hcur, A, Bt = pl.pallas_call(
        _enc_body,
        grid=grid_n,
        in_specs=[
            _bspec((NBLK, 64)), _bspec((NBLK, 16)),
            _wspec((64, 128)), _wspec((1, 128)), _wspec((1, 128)),
            _wspec((1, 128)), _wspec((128, 128)), _wspec((128, 128)),
        ],
        out_specs=[_bspec((NBLK, 128)), _bspec((NBLK, TW)), _bspec((NBLK, TW))],
        out_shape=[
            jax.ShapeDtypeStruct((NPAD, 128), _f32),
            jax.ShapeDtypeStruct((NPAD, TW), _f32),
            jax.ShapeDtypeStruct((NPAD, TW), _f32),
        ],
    )(hp, x16, new_p, r2(p['ne_b']), r2(p['ne_g']), r2(p['ne_beta']), wa0, wb0)

    ea = pl.pallas_call(
        _edge_enc_body,
        grid=grid_e,
        in_specs=[_bspec((EBLK, 16)), _wspec((16, 128)), _wspec((1, 128))],
        out_specs=_bspec((EBLK, 128)),
        out_shape=jax.ShapeDtypeStruct((EPAD, 128), _f32),
    )(eap, eew_p, r2(p['ee_b']))

    for i in range(L):
        lp = lw[i]
        has_coord = i < L - 1
        wc = lp['ew1'][256:257]
        wd = lp['ew1'][257:385]

        av = _gather(A, rowp)
        bv = _gather(Bt, colp)

        if has_coord:
            cw1, cb1, cw2 = lp['cw1'], r2(lp['cb1']), lp['cw2']
            n_out = 2
        else:
            cw1 = jnp.zeros((128, 128), _f32)
            cb1 = jnp.zeros((1, 128), _f32)
            cw2 = jnp.zeros((128, 1), _f32)
            n_out = 1
        eouts = pl.pallas_call(
            functools.partial(_edge_body, has_coord),
            grid=grid_e,
            in_specs=[
                _bspec((EBLK, TW)), _bspec((EBLK, TW)), _bspec((EBLK, 128)),
                _wspec((128, 128)), _wspec((1, 128)), _wspec((1, 128)),
                _wspec((128, 128)), _wspec((1, 128)),
                _wspec((128, 128)), _wspec((1, 128)), _wspec((128, 1)),
            ],
            out_specs=[_bspec((EBLK, SW))] * n_out,
            out_shape=[jax.ShapeDtypeStruct((EPAD, SW), _f32)] * n_out,
        )(av, bv, ea, wd, wc, r2(lp['eb1']), lp['ew2'], r2(lp['eb2']),
          cw1, cb1, cw2)
        if has_coord:
            mv, tv = eouts
        else:
            mv, = eouts

        parts_m = _scatter(rowp, mv, zeros_acc)
        if has_coord:
            parts_t = _scatter(rowp, tv, zeros_acc)

        nw1h = lp['nw1'][0:128]
        nw1m = lp['nw1'][128:256]
        if has_coord:
            wan = lw[i + 1]['ew1'][0:128]
            wbn = lw[i + 1]['ew1'][128:256]
            hcur, x16, A, Bt = pl.pallas_call(
                functools.partial(_node_body, True),
                grid=grid_n,
                in_specs=[
                    _bspec((NBLK, 128)), _bspec((NBLK, 16)),
                    _bspec((NBLK, SW)),
                    pl.BlockSpec((NBLK, SW), lambda j: (NPB + j, 0)),
                    _bspec((NBLK, SW)),
                    pl.BlockSpec((NBLK, SW), lambda j: (NPB + j, 0)),
                    _wspec((128, 128)), _wspec((128, 128)), _wspec((1, 128)),
                    _wspec((128, 128)), _wspec((1, 128)),
                    _wspec((1, 128)), _wspec((1, 128)),
                    _wspec((128, 128)), _wspec((128, 128)),
                ],
                out_specs=[_bspec((NBLK, 128)), _bspec((NBLK, 16)),
                           _bspec((NBLK, TW)), _bspec((NBLK, TW))],
                out_shape=[
                    jax.ShapeDtypeStruct((NPAD, 128), _f32),
                    jax.ShapeDtypeStruct((NPAD, 16), _f32),
                    jax.ShapeDtypeStruct((NPAD, TW), _f32),
                    jax.ShapeDtypeStruct((NPAD, TW), _f32),
                ],
            )(hcur, x16, parts_m, parts_m, parts_t, parts_t,
              nw1h, nw1m, r2(lp['nb1']), lp['nw2'],
              r2(lp['nb2']), r2(lp['ln_g']), r2(lp['ln_b']), wan, wbn)
        else:
            hcur = pl.pallas_call(
                functools.partial(_node_body, False),
                grid=grid_n,
                in_specs=[
                    _bspec((NBLK, 128)),
                    _bspec((NBLK, SW)),
                    pl.BlockSpec((NBLK, SW), lambda j: (NPB + j, 0)),
                    _wspec((128, 128)), _wspec((128, 128)), _wspec((1, 128)),
                    _wspec((128, 128)), _wspec((1, 128)),
                    _wspec((1, 128)), _wspec((1, 128)),
                ],
                out_specs=_bspec((NBLK, 128)),
                out_shape=jax.ShapeDtypeStruct((NPAD, 128), _f32),
            )(hcur, parts_m, parts_m, nw1h, nw1m, r2(lp['nb1']), lp['nw2'],
              r2(lp['nb2']), r2(lp['ln_g']), r2(lp['ln_b']))

    out = pl.pallas_call(
        _pool_body,
        out_shape=jax.ShapeDtypeStruct((B, 1), _f32),
    )(hcur, bp, p['pw1'], r2(p['pb1']), p['pw2'], r2(p['pb2']),
      p['cw1'], r2(p['cb1']), p['cw2'], r2(p['cb2']), p['cw3'], r2(p['cb3']))
    return out


# speedup vs baseline: 1.2817x; 1.2817x over previous
import functools

import jax
import jax.numpy as jnp
from jax import lax
from jax.experimental import pallas as pl
from jax.experimental.pallas import tpu as pltpu
from jax.experimental.pallas import tpu_sc as plsc

N = 10000
E = 320000
B = 64
H = 128
L = 4

NPAD = 10240
NCORE = 2
NSUB = 16
CHUNK = 128
CPT = 80
PER_TILE = CHUNK * CPT
EPAD = NCORE * NSUB * PER_TILE
TW = 128
SW = 128
EBLK = 1024
NBLK = 1024
NPB = NPAD // NBLK

_f32 = jnp.float32
_bf16 = jnp.bfloat16


def _silu(t):
    return t * jax.nn.sigmoid(t)


def _ln(t, g, b):
    mu = jnp.mean(t, -1, keepdims=True)
    d = t - mu
    var = jnp.mean(d * d, -1, keepdims=True)
    return d / jnp.sqrt(var + 1e-5) * g + b


def _wspec(shape):
    nd = len(shape)
    return pl.BlockSpec(shape, lambda i: (0,) * nd)


def _bspec(shape):
    return pl.BlockSpec(shape, lambda i: (i,) + (0,) * (len(shape) - 1))


_u32 = jnp.uint32
_u16 = jnp.uint16
_i32 = jnp.int32


def _pack_proj(hw):
    u = lax.bitcast_convert_type(hw.astype(_bf16), _u16)
    lo = u[:, 0:64].astype(_u32)
    hi = u[:, 64:128].astype(_u32)
    return lax.bitcast_convert_type(lo | (hi << 16), _i32)


def _unpack_proj(w):
    wu = lax.bitcast_convert_type(w, _u32)
    lo = lax.bitcast_convert_type((wu & 0xFFFF).astype(_u16), _bf16)
    hi = lax.bitcast_convert_type((wu >> 16).astype(_u16), _bf16)
    return jnp.concatenate([lo.astype(_f32), hi.astype(_f32)], axis=1)


def _tables(h, x16, wa, wb):
    xi = lax.bitcast_convert_type(x16, _i32)
    z = jnp.zeros((h.shape[0], TW - 80), _i32)
    pa = _pack_proj(jnp.dot(h, wa[...], preferred_element_type=_f32))
    pb = _pack_proj(jnp.dot(h, wb[...], preferred_element_type=_f32))
    a = jnp.concatenate([pa, xi, z], axis=1)
    b = jnp.concatenate([pb, xi, z], axis=1)
    return a, b


def _enc_body(hin, x16, new, neb, neg, nebeta, wa, wb, h_out, a_out, b_out):
    hp = _silu(jnp.dot(hin[...], new[...], preferred_element_type=_f32) + neb[...])
    h = _ln(hp, neg[...], nebeta[...])
    h_out[...] = h
    a_out[...], b_out[...] = _tables(h, x16[...], wa, wb)


def _edge_enc_body(eap, eew, eeb, out):
    out[...] = _silu(jnp.dot(eap[...], eew[...], preferred_element_type=_f32)
                     + eeb[...])


def _edge_body(has_coord, av, bv, ea, wd, wc, eb1, ew2, eb2, cw1, cb1, cw2,
               m_out, t_out=None):
    a = av[...]
    b = bv[...]
    hsum = _unpack_proj(a[:, 0:64]) + _unpack_proj(b[:, 0:64])
    xd16 = (lax.bitcast_convert_type(a[:, 64:80], _f32)
            - lax.bitcast_convert_type(b[:, 64:80], _f32))
    dist_sq = jnp.sum(xd16 * xd16, axis=1, keepdims=True)
    e1 = (hsum
          + jnp.dot(ea[...], wd[...], preferred_element_type=_f32)
          + dist_sq * wc[...] + eb1[...])
    m = _silu(jnp.dot(_silu(e1), ew2[...], preferred_element_type=_f32) + eb2[...])
    m_out[...] = m
    if has_coord:
        lane = lax.broadcasted_iota(jnp.int32, (EBLK, 16), 1)
        t = _silu(jnp.dot(m, cw1[...], preferred_element_type=_f32) + cb1[...])
        c = jnp.tanh(jnp.dot(t, cw2[...], preferred_element_type=_f32))
        dist = jnp.sqrt(dist_sq + 1e-8)
        tail = xd16 * (c / dist)
        tail = jnp.where(lane == 3, 1.0, tail)
        t_out[...] = jnp.concatenate(
            [tail, jnp.zeros((EBLK, SW - 16), _f32)], axis=1)


def _node_body(has_coord, *refs):
    if has_coord:
        (h_in, x16, p0, p1, t0, t1, nw1h, nw1m, nb1, nw2, nb2, lng, lnb,
         wa, wb, h_out, x_out, a_out, b_out) = refs
    else:
        h_in, p0, p1, nw1h, nw1m, nb1, nw2, nb2, lng, lnb, h_out = refs
    h = h_in[...]
    m_i = p0[:, 0:128] + p1[:, 0:128]
    hu = _silu(jnp.dot(h, nw1h[...], preferred_element_type=_f32)
               + jnp.dot(m_i, nw1m[...], preferred_element_type=_f32)
               + nb1[...])
    hu = jnp.dot(hu, nw2[...], preferred_element_type=_f32) + nb2[...]
    hn = _ln(h + hu, lng[...], lnb[...])
    h_out[...] = hn
    if has_coord:
        tail = t0[:, 0:16] + t1[:, 0:16]
        deg = jnp.maximum(tail[:, 3:4], 1.0)
        lane = lax.broadcasted_iota(jnp.int32, (NBLK, 16), 1)
        xn = x16[...] + jnp.where(lane < 3, tail, 0.0) / deg
        x_out[...] = xn
        a_out[...], b_out[...] = _tables(hn, xn, wa, wb)


def _pool_body(h_ref, bp_ref, pw1, pb1, pw2, pb2, cw1, cb1, cw2, cb2, cw3, cb3,
               out_ref):
    h = h_ref[...]
    bp = bp_ref[...]
    s = jnp.dot(jnp.tanh(jnp.dot(h, pw1[...], preferred_element_type=_f32)
                         + pb1[...]),
                pw2[...], preferred_element_type=_f32) + pb2[...]
    cols = lax.broadcasted_iota(jnp.int32, (NPAD, B), 1)
    m = bp == cols
    mf = m.astype(_f32)
    dn = (((0,), (0,)), ((), ()))
    smax = jnp.max(jnp.where(m, s, -1e30), axis=0, keepdims=True)
    sg = jnp.sum(jnp.where(m, smax, 0.0), axis=1, keepdims=True)
    sexp = jnp.exp(s - sg)
    ssum = lax.dot_general(sexp, mf, dn, preferred_element_type=_f32)
    sden = jnp.sum(jnp.where(m, ssum, 0.0), axis=1, keepdims=True)
    w = jnp.where(bp >= 0, sexp / (sden + 1e-16), 0.0)
    g = lax.dot_general(mf, h * w, dn, preferred_element_type=_f32)
    inv = 0.9999950000374996
    z = _silu(jnp.dot(g, cw1[...], preferred_element_type=_f32) + cb1[...]) * inv
    z = _silu(jnp.dot(z, cw2[...], preferred_element_type=_f32) + cb2[...]) * inv
    out_ref[...] = jnp.dot(z, cw3[...], preferred_element_type=_f32) + cb3[...]


def _gather_body(ta, tb, rows_hbm, cols_hbm, oa, ob, idxa, idxb,
                 a0, a1, b0, b1, sga0, sga1, sgb0, sgb1,
                 swa0, swa1, swb0, swb1):
    wid = lax.axis_index("c") * NSUB + lax.axis_index("s")
    cbase = wid * CPT
    ebase = cbase * CHUNK
    pltpu.sync_copy(rows_hbm.at[pl.ds(cbase, CPT)], idxa)
    pltpu.sync_copy(cols_hbm.at[pl.ds(cbase, CPT)], idxb)
    abufs = (a0, a1)
    bbufs = (b0, b1)
    sgas = (sga0, sga1)
    sgbs = (sgb0, sgb1)
    swas = (swa0, swa1)
    swbs = (swb0, swb1)

    pltpu.async_copy(ta.at[idxa.at[0]], a0, sga0)
    pltpu.async_copy(tb.at[idxb.at[0]], b0, sgb0)

    @pl.loop(0, CPT, step=2)
    def _pipe(t0):
        for par in range(2):
            t = t0 + par
            ab, bb = abufs[par], bbufs[par]
            sga, sgb = sgas[par], sgbs[par]
            swa, swb = swas[par], swbs[par]
            oab, obb = abufs[1 - par], bbufs[1 - par]
            osga, osgb = sgas[1 - par], sgbs[1 - par]
            oswa, oswb = swas[1 - par], swbs[1 - par]

            @pl.when(t >= 1)
            def _():
                off = ebase + (t - 1) * CHUNK
                pltpu.make_async_copy(oab, oa.at[pl.ds(off, CHUNK)],
                                      oswa).wait()
                pltpu.make_async_copy(obb, ob.at[pl.ds(off, CHUNK)],
                                      oswb).wait()

            @pl.when(t + 1 < CPT)
            def _():
                pltpu.async_copy(ta.at[idxa.at[t + 1]], oab, osga)
                pltpu.async_copy(tb.at[idxb.at[t + 1]], obb, osgb)

            off = ebase + t * CHUNK
            pltpu.make_async_copy(ta.at[idxa.at[t]], ab, sga).wait()
            pltpu.async_copy(ab, oa.at[pl.ds(off, CHUNK)], swa)
            pltpu.make_async_copy(tb.at[idxb.at[t]], bb, sgb).wait()
            pltpu.async_copy(bb, ob.at[pl.ds(off, CHUNK)], swb)

    offl = ebase + (CPT - 1) * CHUNK
    pltpu.make_async_copy(a1, oa.at[pl.ds(offl, CHUNK)], swa1).wait()
    pltpu.make_async_copy(b1, ob.at[pl.ds(offl, CHUNK)], swb1).wait()


_gather = pl.kernel(
    _gather_body,
    out_type=[
        jax.ShapeDtypeStruct((EPAD, TW), _i32),
        jax.ShapeDtypeStruct((EPAD, TW), _i32),
    ],
    mesh=plsc.VectorSubcoreMesh(core_axis_name="c", subcore_axis_name="s"),
    scratch_types=[
        pltpu.VMEM((CPT, CHUNK), jnp.int32),
        pltpu.VMEM((CPT, CHUNK), jnp.int32),
        pltpu.VMEM((CHUNK, TW), _i32),
        pltpu.VMEM((CHUNK, TW), _i32),
        pltpu.VMEM((CHUNK, TW), _i32),
        pltpu.VMEM((CHUNK, TW), _i32),
    ] + [pltpu.SemaphoreType.DMA] * 8,
)


def _scatter_body(idx_hbm, mv_hbm, zero_hbm, out_hbm, idx2d, buf0, buf1,
                  acc_sh, sv0, sv1, ss0, ss1):
    cid = lax.axis_index("c")
    sid = lax.axis_index("s")
    wid = cid * NSUB + sid
    rows = NPAD // NSUB
    rbase = sid * rows
    cbase = wid * CPT
    ebase = cbase * CHUNK
    pltpu.sync_copy(idx_hbm.at[pl.ds(cbase, CPT)], idx2d)
    pltpu.sync_copy(zero_hbm.at[pl.ds(rbase, rows)],
                    acc_sh.at[pl.ds(rbase, rows)])
    plsc.subcore_barrier()

    bufs = (buf0, buf1)
    svs = (sv0, sv1)
    sss = (ss0, ss1)

    pltpu.async_copy(mv_hbm.at[pl.ds(ebase, CHUNK)], buf0, sv0)

    @pl.loop(0, CPT, step=2)
    def _pipe(t0):
        for bpar in range(2):
            t = t0 + bpar
            buf, sv, ss = bufs[bpar], svs[bpar], sss[bpar]
            obuf, osv, oss = bufs[1 - bpar], svs[1 - bpar], sss[1 - bpar]

            @pl.when(t + 1 < CPT)
            def _():
                @pl.when(t >= 1)
                def _():
                    pltpu.make_async_copy(
                        obuf, acc_sh.at[idx2d.at[t - 1]], oss).wait()
                pltpu.async_copy(
                    mv_hbm.at[pl.ds(ebase + (t + 1) * CHUNK, CHUNK)],
                    obuf, osv)

            pltpu.make_async_copy(
                mv_hbm.at[pl.ds(ebase + t * CHUNK, CHUNK)], buf, sv).wait()
            pltpu.async_copy(buf, acc_sh.at[idx2d.at[t]], ss, add=True)

    pltpu.make_async_copy(buf0, acc_sh.at[idx2d.at[CPT - 2]], ss0).wait()
    pltpu.make_async_copy(buf1, acc_sh.at[idx2d.at[CPT - 1]], ss1).wait()
    plsc.subcore_barrier()
    obase = cid * NPAD + rbase
    pltpu.sync_copy(acc_sh.at[pl.ds(rbase, rows)],
                    out_hbm.at[pl.ds(obase, rows)])


_scatter = pl.kernel(
    _scatter_body,
    out_type=jax.ShapeDtypeStruct((NCORE * NPAD, SW), _f32),
    mesh=plsc.VectorSubcoreMesh(core_axis_name="c", subcore_axis_name="s"),
    scratch_types=[
        pltpu.VMEM((CPT, CHUNK), jnp.int32),
        pltpu.VMEM((CHUNK, SW), _f32),
        pltpu.VMEM((CHUNK, SW), _f32),
        pltpu.VMEM_SHARED((NPAD, SW), _f32),
        pltpu.SemaphoreType.DMA,
        pltpu.SemaphoreType.DMA,
        pltpu.SemaphoreType.DMA,
        pltpu.SemaphoreType.DMA,
    ],
)


def kernel(h, x, edge_index, edge_attr, batch, params):
    p = params
    r2 = lambda t: t.reshape(1, -1)

    hp = jnp.pad(h, ((0, NPAD - N), (0, 64 - 58)))
    x16 = jnp.pad(x, ((0, NPAD - N), (0, 13)))
    eap = jnp.pad(edge_attr, ((0, EPAD - E), (0, 4)))
    rowp = jnp.pad(edge_index[0], (0, EPAD - E),
                   constant_values=NPAD - 1).reshape(EPAD // CHUNK, CHUNK)
    colp = jnp.pad(edge_index[1], (0, EPAD - E),
                   constant_values=NPAD - 1).reshape(EPAD // CHUNK, CHUNK)
    bp = jnp.pad(batch, (0, NPAD - N), constant_values=-1).reshape(NPAD, 1)
    zeros_acc = jnp.zeros((NPAD, SW), _f32)
    new_p = jnp.pad(p['ne_w'], ((0, 6), (0, 0)))
    eew_p = jnp.pad(p['ee_w'], ((0, 4), (0, 0)))

    lw = p['layers']
    wa0 = lw[0]['ew1'][0:128]
    wb0 = lw[0]['ew1'][128:256]

    grid_n = (NPB,)
    grid_e = (EPAD // EBLK,)

    hcur, A, Bt = pl.pallas_call(
        _enc_body,
        grid=grid_n,
        in_specs=[
            _bspec((NBLK, 64)), _bspec((NBLK, 16)),
            _wspec((64, 128)), _wspec((1, 128)), _wspec((1, 128)),
            _wspec((1, 128)), _wspec((128, 128)), _wspec((128, 128)),
        ],
        out_specs=[_bspec((NBLK, 128)), _bspec((NBLK, TW)), _bspec((NBLK, TW))],
        out_shape=[
            jax.ShapeDtypeStruct((NPAD, 128), _f32),
            jax.ShapeDtypeStruct((NPAD, TW), _i32),
            jax.ShapeDtypeStruct((NPAD, TW), _i32),
        ],
    )(hp, x16, new_p, r2(p['ne_b']), r2(p['ne_g']), r2(p['ne_beta']), wa0, wb0)

    ea = pl.pallas_call(
        _edge_enc_body,
        grid=grid_e,
        in_specs=[_bspec((EBLK, 16)), _wspec((16, 128)), _wspec((1, 128))],
        out_specs=_bspec((EBLK, 128)),
        out_shape=jax.ShapeDtypeStruct((EPAD, 128), _f32),
    )(eap, eew_p, r2(p['ee_b']))

    for i in range(L):
        lp = lw[i]
        has_coord = i < L - 1
        wc = lp['ew1'][256:257]
        wd = lp['ew1'][257:385]

        av, bv = _gather(A, Bt, rowp, colp)

        if has_coord:
            cw1, cb1, cw2 = lp['cw1'], r2(lp['cb1']), lp['cw2']
            n_out = 2
        else:
            cw1 = jnp.zeros((128, 128), _f32)
            cb1 = jnp.zeros((1, 128), _f32)
            cw2 = jnp.zeros((128, 1), _f32)
            n_out = 1
        eouts = pl.pallas_call(
            functools.partial(_edge_body, has_coord),
            grid=grid_e,
            in_specs=[
                _bspec((EBLK, TW)), _bspec((EBLK, TW)), _bspec((EBLK, 128)),
                _wspec((128, 128)), _wspec((1, 128)), _wspec((1, 128)),
                _wspec((128, 128)), _wspec((1, 128)),
                _wspec((128, 128)), _wspec((1, 128)), _wspec((128, 1)),
            ],
            out_specs=[_bspec((EBLK, SW))] * n_out,
            out_shape=[jax.ShapeDtypeStruct((EPAD, SW), _f32)] * n_out,
        )(av, bv, ea, wd, wc, r2(lp['eb1']), lp['ew2'], r2(lp['eb2']),
          cw1, cb1, cw2)
        if has_coord:
            mv, tv = eouts
        else:
            mv, = eouts

        parts_m = _scatter(rowp, mv, zeros_acc)
        if has_coord:
            parts_t = _scatter(rowp, tv, zeros_acc)

        nw1h = lp['nw1'][0:128]
        nw1m = lp['nw1'][128:256]
        if has_coord:
            wan = lw[i + 1]['ew1'][0:128]
            wbn = lw[i + 1]['ew1'][128:256]
            hcur, x16, A, Bt = pl.pallas_call(
                functools.partial(_node_body, True),
                grid=grid_n,
                in_specs=[
                    _bspec((NBLK, 128)), _bspec((NBLK, 16)),
                    _bspec((NBLK, SW)),
                    pl.BlockSpec((NBLK, SW), lambda j: (NPB + j, 0)),
                    _bspec((NBLK, SW)),
                    pl.BlockSpec((NBLK, SW), lambda j: (NPB + j, 0)),
                    _wspec((128, 128)), _wspec((128, 128)), _wspec((1, 128)),
                    _wspec((128, 128)), _wspec((1, 128)),
                    _wspec((1, 128)), _wspec((1, 128)),
                    _wspec((128, 128)), _wspec((128, 128)),
                ],
                out_specs=[_bspec((NBLK, 128)), _bspec((NBLK, 16)),
                           _bspec((NBLK, TW)), _bspec((NBLK, TW))],
                out_shape=[
                    jax.ShapeDtypeStruct((NPAD, 128), _f32),
                    jax.ShapeDtypeStruct((NPAD, 16), _f32),
                    jax.ShapeDtypeStruct((NPAD, TW), _i32),
                    jax.ShapeDtypeStruct((NPAD, TW), _i32),
                ],
            )(hcur, x16, parts_m, parts_m, parts_t, parts_t,
              nw1h, nw1m, r2(lp['nb1']), lp['nw2'],
              r2(lp['nb2']), r2(lp['ln_g']), r2(lp['ln_b']), wan, wbn)
        else:
            hcur = pl.pallas_call(
                functools.partial(_node_body, False),
                grid=grid_n,
                in_specs=[
                    _bspec((NBLK, 128)),
                    _bspec((NBLK, SW)),
                    pl.BlockSpec((NBLK, SW), lambda j: (NPB + j, 0)),
                    _wspec((128, 128)), _wspec((128, 128)), _wspec((1, 128)),
                    _wspec((128, 128)), _wspec((1, 128)),
                    _wspec((1, 128)), _wspec((1, 128)),
                ],
                out_specs=_bspec((NBLK, 128)),
                out_shape=jax.ShapeDtypeStruct((NPAD, 128), _f32),
            )(hcur, parts_m, parts_m, nw1h, nw1m, r2(lp['nb1']), lp['nw2'],
              r2(lp['nb2']), r2(lp['ln_g']), r2(lp['ln_b']))

    out = pl.pallas_call(
        _pool_body,
        out_shape=jax.ShapeDtypeStruct((B, 1), _f32),
    )(hcur, bp, p['pw1'], r2(p['pb1']), p['pw2'], r2(p['pb2']),
      p['cw1'], r2(p['cb1']), p['cw2'], r2(p['cb2']), p['cw3'], r2(p['cb3']))
    return out


# speedup vs baseline: 1.3295x; 1.0373x over previous
import functools

import jax
import jax.numpy as jnp
from jax import lax
from jax.experimental import pallas as pl
from jax.experimental.pallas import tpu as pltpu
from jax.experimental.pallas import tpu_sc as plsc

N = 10000
E = 320000
B = 64
H = 128
L = 4

NPAD = 10240
NCORE = 2
NSUB = 16
CHUNK = 128
CPT = 80
PER_TILE = CHUNK * CPT
EPAD = NCORE * NSUB * PER_TILE
TW = 128
SW = 128
EBLK = 1024
NBLK = 1024
NPB = NPAD // NBLK

_f32 = jnp.float32
_bf16 = jnp.bfloat16


def _silu(t):
    return t * jax.nn.sigmoid(t)


def _ln(t, g, b):
    mu = jnp.mean(t, -1, keepdims=True)
    d = t - mu
    var = jnp.mean(d * d, -1, keepdims=True)
    return d / jnp.sqrt(var + 1e-5) * g + b


def _wspec(shape):
    nd = len(shape)
    return pl.BlockSpec(shape, lambda i: (0,) * nd)


def _bspec(shape):
    return pl.BlockSpec(shape, lambda i: (i,) + (0,) * (len(shape) - 1))


_u32 = jnp.uint32
_u16 = jnp.uint16
_i32 = jnp.int32


def _pack_proj(hw):
    u = lax.bitcast_convert_type(hw.astype(_bf16), _u16)
    lo = u[:, 0:64].astype(_u32)
    hi = u[:, 64:128].astype(_u32)
    return lax.bitcast_convert_type(lo | (hi << 16), _i32)


def _unpack_proj(w):
    wu = lax.bitcast_convert_type(w, _u32)
    lo = lax.bitcast_convert_type((wu & 0xFFFF).astype(_u16), _bf16)
    hi = lax.bitcast_convert_type((wu >> 16).astype(_u16), _bf16)
    return jnp.concatenate([lo.astype(_f32), hi.astype(_f32)], axis=1)


def _tables(h, x16, wa, wb):
    xi = lax.bitcast_convert_type(x16, _i32)
    z = jnp.zeros((h.shape[0], TW - 80), _i32)
    pa = _pack_proj(jnp.dot(h, wa[...], preferred_element_type=_f32))
    pb = _pack_proj(jnp.dot(h, wb[...], preferred_element_type=_f32))
    a = jnp.concatenate([pa, xi, z], axis=1)
    b = jnp.concatenate([pb, xi, z], axis=1)
    return a, b


def _enc_body(hin, x16, new, neb, neg, nebeta, wa, wb, h_out, a_out, b_out):
    hp = _silu(jnp.dot(hin[...], new[...], preferred_element_type=_f32) + neb[...])
    h = _ln(hp, neg[...], nebeta[...])
    h_out[...] = h
    a_out[...], b_out[...] = _tables(h, x16[...], wa, wb)


def _edge_enc_body(eap, eew, eeb, out):
    out[...] = _silu(jnp.dot(eap[...], eew[...], preferred_element_type=_f32)
                     + eeb[...]).astype(_bf16)


def _edge_body(has_coord, av, bv, ea, wd, wc, eb1, ew2, eb2, cw1, cb1, cw2,
               m_out, t_out=None):
    a = av[...]
    b = bv[...]
    hsum = _unpack_proj(a[:, 0:64]) + _unpack_proj(b[:, 0:64])
    xd16 = (lax.bitcast_convert_type(a[:, 64:80], _f32)
            - lax.bitcast_convert_type(b[:, 64:80], _f32))
    dist_sq = jnp.sum(xd16 * xd16, axis=1, keepdims=True)
    e1 = (hsum
          + jnp.dot(ea[...].astype(_f32), wd[...], preferred_element_type=_f32)
          + dist_sq * wc[...] + eb1[...])
    m = _silu(jnp.dot(_silu(e1), ew2[...], preferred_element_type=_f32) + eb2[...])
    m_out[...] = m
    if has_coord:
        lane = lax.broadcasted_iota(jnp.int32, (EBLK, 16), 1)
        t = _silu(jnp.dot(m, cw1[...], preferred_element_type=_f32) + cb1[...])
        c = jnp.tanh(jnp.dot(t, cw2[...], preferred_element_type=_f32))
        dist = jnp.sqrt(dist_sq + 1e-8)
        tail = xd16 * (c / dist)
        tail = jnp.where(lane == 3, 1.0, tail)
        t_out[...] = jnp.concatenate(
            [tail, jnp.zeros((EBLK, SW - 16), _f32)], axis=1)


def _node_body(has_coord, *refs):
    if has_coord:
        (h_in, x16, p0, p1, t0, t1, nw1h, nw1m, nb1, nw2, nb2, lng, lnb,
         wa, wb, h_out, x_out, a_out, b_out) = refs
    else:
        h_in, p0, p1, nw1h, nw1m, nb1, nw2, nb2, lng, lnb, h_out = refs
    h = h_in[...]
    m_i = p0[:, 0:128] + p1[:, 0:128]
    hu = _silu(jnp.dot(h, nw1h[...], preferred_element_type=_f32)
               + jnp.dot(m_i, nw1m[...], preferred_element_type=_f32)
               + nb1[...])
    hu = jnp.dot(hu, nw2[...], preferred_element_type=_f32) + nb2[...]
    hn = _ln(h + hu, lng[...], lnb[...])
    h_out[...] = hn
    if has_coord:
        tail = t0[:, 0:16] + t1[:, 0:16]
        deg = jnp.maximum(tail[:, 3:4], 1.0)
        lane = lax.broadcasted_iota(jnp.int32, (NBLK, 16), 1)
        xn = x16[...] + jnp.where(lane < 3, tail, 0.0) / deg
        x_out[...] = xn
        a_out[...], b_out[...] = _tables(hn, xn, wa, wb)


def _pool_body(h_ref, bp_ref, pw1, pb1, pw2, pb2, cw1, cb1, cw2, cb2, cw3, cb3,
               out_ref):
    h = h_ref[...]
    bp = bp_ref[...]
    s = jnp.dot(jnp.tanh(jnp.dot(h, pw1[...], preferred_element_type=_f32)
                         + pb1[...]),
                pw2[...], preferred_element_type=_f32) + pb2[...]
    cols = lax.broadcasted_iota(jnp.int32, (NPAD, B), 1)
    m = bp == cols
    mf = m.astype(_f32)
    dn = (((0,), (0,)), ((), ()))
    smax = jnp.max(jnp.where(m, s, -1e30), axis=0, keepdims=True)
    sg = jnp.sum(jnp.where(m, smax, 0.0), axis=1, keepdims=True)
    sexp = jnp.exp(s - sg)
    ssum = lax.dot_general(sexp, mf, dn, preferred_element_type=_f32)
    sden = jnp.sum(jnp.where(m, ssum, 0.0), axis=1, keepdims=True)
    w = jnp.where(bp >= 0, sexp / (sden + 1e-16), 0.0)
    g = lax.dot_general(mf, h * w, dn, preferred_element_type=_f32)
    inv = 0.9999950000374996
    z = _silu(jnp.dot(g, cw1[...], preferred_element_type=_f32) + cb1[...]) * inv
    z = _silu(jnp.dot(z, cw2[...], preferred_element_type=_f32) + cb2[...]) * inv
    out_ref[...] = jnp.dot(z, cw3[...], preferred_element_type=_f32) + cb3[...]


def _gather_body(ta, tb, rows_hbm, cols_hbm, oa, ob, idxa, idxb,
                 a0, a1, b0, b1, sga0, sga1, sgb0, sgb1,
                 swa0, swa1, swb0, swb1):
    wid = lax.axis_index("c") * NSUB + lax.axis_index("s")
    cbase = wid * CPT
    ebase = cbase * CHUNK
    pltpu.sync_copy(rows_hbm.at[pl.ds(cbase, CPT)], idxa)
    pltpu.sync_copy(cols_hbm.at[pl.ds(cbase, CPT)], idxb)
    abufs = (a0, a1)
    bbufs = (b0, b1)
    sgas = (sga0, sga1)
    sgbs = (sgb0, sgb1)
    swas = (swa0, swa1)
    swbs = (swb0, swb1)

    pltpu.async_copy(ta.at[idxa.at[0]], a0, sga0)
    pltpu.async_copy(tb.at[idxb.at[0]], b0, sgb0)

    @pl.loop(0, CPT, step=2)
    def _pipe(t0):
        for par in range(2):
            t = t0 + par
            ab, bb = abufs[par], bbufs[par]
            sga, sgb = sgas[par], sgbs[par]
            swa, swb = swas[par], swbs[par]
            oab, obb = abufs[1 - par], bbufs[1 - par]
            osga, osgb = sgas[1 - par], sgbs[1 - par]
            oswa, oswb = swas[1 - par], swbs[1 - par]

            @pl.when(t >= 1)
            def _():
                off = ebase + (t - 1) * CHUNK
                pltpu.make_async_copy(oab, oa.at[pl.ds(off, CHUNK)],
                                      oswa).wait()
                pltpu.make_async_copy(obb, ob.at[pl.ds(off, CHUNK)],
                                      oswb).wait()

            @pl.when(t + 1 < CPT)
            def _():
                pltpu.async_copy(ta.at[idxa.at[t + 1]], oab, osga)
                pltpu.async_copy(tb.at[idxb.at[t + 1]], obb, osgb)

            off = ebase + t * CHUNK
            pltpu.make_async_copy(ta.at[idxa.at[t]], ab, sga).wait()
            pltpu.async_copy(ab, oa.at[pl.ds(off, CHUNK)], swa)
            pltpu.make_async_copy(tb.at[idxb.at[t]], bb, sgb).wait()
            pltpu.async_copy(bb, ob.at[pl.ds(off, CHUNK)], swb)

    offl = ebase + (CPT - 1) * CHUNK
    pltpu.make_async_copy(a1, oa.at[pl.ds(offl, CHUNK)], swa1).wait()
    pltpu.make_async_copy(b1, ob.at[pl.ds(offl, CHUNK)], swb1).wait()


_gather = pl.kernel(
    _gather_body,
    out_type=[
        jax.ShapeDtypeStruct((EPAD, TW), _i32),
        jax.ShapeDtypeStruct((EPAD, TW), _i32),
    ],
    mesh=plsc.VectorSubcoreMesh(core_axis_name="c", subcore_axis_name="s"),
    scratch_types=[
        pltpu.VMEM((CPT, CHUNK), jnp.int32),
        pltpu.VMEM((CPT, CHUNK), jnp.int32),
        pltpu.VMEM((CHUNK, TW), _i32),
        pltpu.VMEM((CHUNK, TW), _i32),
        pltpu.VMEM((CHUNK, TW), _i32),
        pltpu.VMEM((CHUNK, TW), _i32),
    ] + [pltpu.SemaphoreType.DMA] * 8,
)


def _scatter_body(idx_hbm, mv_hbm, zero_hbm, out_hbm, idx2d, buf0, buf1,
                  acc_sh, sv0, sv1, ss0, ss1):
    cid = lax.axis_index("c")
    sid = lax.axis_index("s")
    wid = cid * NSUB + sid
    rows = NPAD // NSUB
    rbase = sid * rows
    cbase = wid * CPT
    ebase = cbase * CHUNK
    pltpu.sync_copy(idx_hbm.at[pl.ds(cbase, CPT)], idx2d)
    pltpu.sync_copy(zero_hbm.at[pl.ds(rbase, rows)],
                    acc_sh.at[pl.ds(rbase, rows)])
    plsc.subcore_barrier()

    bufs = (buf0, buf1)
    svs = (sv0, sv1)
    sss = (ss0, ss1)

    pltpu.async_copy(mv_hbm.at[pl.ds(ebase, CHUNK)], buf0, sv0)

    @pl.loop(0, CPT, step=2)
    def _pipe(t0):
        for bpar in range(2):
            t = t0 + bpar
            buf, sv, ss = bufs[bpar], svs[bpar], sss[bpar]
            obuf, osv, oss = bufs[1 - bpar], svs[1 - bpar], sss[1 - bpar]

            @pl.when(t + 1 < CPT)
            def _():
                @pl.when(t >= 1)
                def _():
                    pltpu.make_async_copy(
                        obuf, acc_sh.at[idx2d.at[t - 1]], oss).wait()
                pltpu.async_copy(
                    mv_hbm.at[pl.ds(ebase + (t + 1) * CHUNK, CHUNK)],
                    obuf, osv)

            pltpu.make_async_copy(
                mv_hbm.at[pl.ds(ebase + t * CHUNK, CHUNK)], buf, sv).wait()
            pltpu.async_copy(buf, acc_sh.at[idx2d.at[t]], ss, add=True)

    pltpu.make_async_copy(buf0, acc_sh.at[idx2d.at[CPT - 2]], ss0).wait()
    pltpu.make_async_copy(buf1, acc_sh.at[idx2d.at[CPT - 1]], ss1).wait()
    plsc.subcore_barrier()
    obase = cid * NPAD + rbase
    pltpu.sync_copy(acc_sh.at[pl.ds(rbase, rows)],
                    out_hbm.at[pl.ds(obase, rows)])


_scatter = pl.kernel(
    _scatter_body,
    out_type=jax.ShapeDtypeStruct((NCORE * NPAD, SW), _f32),
    mesh=plsc.VectorSubcoreMesh(core_axis_name="c", subcore_axis_name="s"),
    scratch_types=[
        pltpu.VMEM((CPT, CHUNK), jnp.int32),
        pltpu.VMEM((CHUNK, SW), _f32),
        pltpu.VMEM((CHUNK, SW), _f32),
        pltpu.VMEM_SHARED((NPAD, SW), _f32),
        pltpu.SemaphoreType.DMA,
        pltpu.SemaphoreType.DMA,
        pltpu.SemaphoreType.DMA,
        pltpu.SemaphoreType.DMA,
    ],
)


def kernel(h, x, edge_index, edge_attr, batch, params):
    p = params
    r2 = lambda t: t.reshape(1, -1)

    hp = jnp.pad(h, ((0, NPAD - N), (0, 64 - 58)))
    x16 = jnp.pad(x, ((0, NPAD - N), (0, 13)))
    eap = jnp.pad(edge_attr, ((0, EPAD - E), (0, 4)))
    rowp = jnp.pad(edge_index[0], (0, EPAD - E),
                   constant_values=NPAD - 1).reshape(EPAD // CHUNK, CHUNK)
    colp = jnp.pad(edge_index[1], (0, EPAD - E),
                   constant_values=NPAD - 1).reshape(EPAD // CHUNK, CHUNK)
    bp = jnp.pad(batch, (0, NPAD - N), constant_values=-1).reshape(NPAD, 1)
    zeros_acc = jnp.zeros((NPAD, SW), _f32)
    new_p = jnp.pad(p['ne_w'], ((0, 6), (0, 0)))
    eew_p = jnp.pad(p['ee_w'], ((0, 4), (0, 0)))

    lw = p['layers']
    wa0 = lw[0]['ew1'][0:128]
    wb0 = lw[0]['ew1'][128:256]

    grid_n = (NPB,)
    grid_e = (EPAD // EBLK,)

    hcur, A, Bt = pl.pallas_call(
        _enc_body,
        grid=grid_n,
        in_specs=[
            _bspec((NBLK, 64)), _bspec((NBLK, 16)),
            _wspec((64, 128)), _wspec((1, 128)), _wspec((1, 128)),
            _wspec((1, 128)), _wspec((128, 128)), _wspec((128, 128)),
        ],
        out_specs=[_bspec((NBLK, 128)), _bspec((NBLK, TW)), _bspec((NBLK, TW))],
        out_shape=[
            jax.ShapeDtypeStruct((NPAD, 128), _f32),
            jax.ShapeDtypeStruct((NPAD, TW), _i32),
            jax.ShapeDtypeStruct((NPAD, TW), _i32),
        ],
    )(hp, x16, new_p, r2(p['ne_b']), r2(p['ne_g']), r2(p['ne_beta']), wa0, wb0)

    ea = pl.pallas_call(
        _edge_enc_body,
        grid=grid_e,
        in_specs=[_bspec((EBLK, 16)), _wspec((16, 128)), _wspec((1, 128))],
        out_specs=_bspec((EBLK, 128)),
        out_shape=jax.ShapeDtypeStruct((EPAD, 128), _bf16),
    )(eap, eew_p, r2(p['ee_b']))

    for i in range(L):
        lp = lw[i]
        has_coord = i < L - 1
        wc = lp['ew1'][256:257]
        wd = lp['ew1'][257:385]

        av, bv = _gather(A, Bt, rowp, colp)

        if has_coord:
            cw1, cb1, cw2 = lp['cw1'], r2(lp['cb1']), lp['cw2']
            n_out = 2
        else:
            cw1 = jnp.zeros((128, 128), _f32)
            cb1 = jnp.zeros((1, 128), _f32)
            cw2 = jnp.zeros((128, 1), _f32)
            n_out = 1
        eouts = pl.pallas_call(
            functools.partial(_edge_body, has_coord),
            grid=grid_e,
            in_specs=[
                _bspec((EBLK, TW)), _bspec((EBLK, TW)), _bspec((EBLK, 128)),
                _wspec((128, 128)), _wspec((1, 128)), _wspec((1, 128)),
                _wspec((128, 128)), _wspec((1, 128)),
                _wspec((128, 128)), _wspec((1, 128)), _wspec((128, 1)),
            ],
            out_specs=[_bspec((EBLK, SW))] * n_out,
            out_shape=[jax.ShapeDtypeStruct((EPAD, SW), _f32)] * n_out,
        )(av, bv, ea, wd, wc, r2(lp['eb1']), lp['ew2'], r2(lp['eb2']),
          cw1, cb1, cw2)
        if has_coord:
            mv, tv = eouts
        else:
            mv, = eouts

        parts_m = _scatter(rowp, mv, zeros_acc)
        if has_coord:
            parts_t = _scatter(rowp, tv, zeros_acc)

        nw1h = lp['nw1'][0:128]
        nw1m = lp['nw1'][128:256]
        if has_coord:
            wan = lw[i + 1]['ew1'][0:128]
            wbn = lw[i + 1]['ew1'][128:256]
            hcur, x16, A, Bt = pl.pallas_call(
                functools.partial(_node_body, True),
                grid=grid_n,
                in_specs=[
                    _bspec((NBLK, 128)), _bspec((NBLK, 16)),
                    _bspec((NBLK, SW)),
                    pl.BlockSpec((NBLK, SW), lambda j: (NPB + j, 0)),
                    _bspec((NBLK, SW)),
                    pl.BlockSpec((NBLK, SW), lambda j: (NPB + j, 0)),
                    _wspec((128, 128)), _wspec((128, 128)), _wspec((1, 128)),
                    _wspec((128, 128)), _wspec((1, 128)),
                    _wspec((1, 128)), _wspec((1, 128)),
                    _wspec((128, 128)), _wspec((128, 128)),
                ],
                out_specs=[_bspec((NBLK, 128)), _bspec((NBLK, 16)),
                           _bspec((NBLK, TW)), _bspec((NBLK, TW))],
                out_shape=[
                    jax.ShapeDtypeStruct((NPAD, 128), _f32),
                    jax.ShapeDtypeStruct((NPAD, 16), _f32),
                    jax.ShapeDtypeStruct((NPAD, TW), _i32),
                    jax.ShapeDtypeStruct((NPAD, TW), _i32),
                ],
            )(hcur, x16, parts_m, parts_m, parts_t, parts_t,
              nw1h, nw1m, r2(lp['nb1']), lp['nw2'],
              r2(lp['nb2']), r2(lp['ln_g']), r2(lp['ln_b']), wan, wbn)
        else:
            hcur = pl.pallas_call(
                functools.partial(_node_body, False),
                grid=grid_n,
                in_specs=[
                    _bspec((NBLK, 128)),
                    _bspec((NBLK, SW)),
                    pl.BlockSpec((NBLK, SW), lambda j: (NPB + j, 0)),
                    _wspec((128, 128)), _wspec((128, 128)), _wspec((1, 128)),
                    _wspec((128, 128)), _wspec((1, 128)),
                    _wspec((1, 128)), _wspec((1, 128)),
                ],
                out_specs=_bspec((NBLK, 128)),
                out_shape=jax.ShapeDtypeStruct((NPAD, 128), _f32),
            )(hcur, parts_m, parts_m, nw1h, nw1m, r2(lp['nb1']), lp['nw2'],
              r2(lp['nb2']), r2(lp['ln_g']), r2(lp['ln_b']))

    out = pl.pallas_call(
        _pool_body,
        out_shape=jax.ShapeDtypeStruct((B, 1), _f32),
    )(hcur, bp, p['pw1'], r2(p['pb1']), p['pw2'], r2(p['pb2']),
      p['cw1'], r2(p['cb1']), p['cw2'], r2(p['cb2']), p['cw3'], r2(p['cb3']))
    return out


# speedup vs baseline: 1.5138x; 1.1386x over previous
import functools

import jax
import jax.numpy as jnp
from jax import lax
from jax.experimental import pallas as pl
from jax.experimental.pallas import tpu as pltpu
from jax.experimental.pallas import tpu_sc as plsc

N = 10000
E = 320000
B = 64
H = 128
L = 4

NPAD = 10240
NACC = 10112
NCORE = 2
NSUB = 16
CHUNK = 128
CPT = 80
PER_TILE = CHUNK * CPT
EPAD = NCORE * NSUB * PER_TILE
TW = 128
SW = 128
EBLK = 1024
NBLK = 1024
NPB = NPAD // NBLK

_f32 = jnp.float32
_bf16 = jnp.bfloat16


def _silu(t):
    return t * jax.nn.sigmoid(t)


def _ln(t, g, b):
    mu = jnp.mean(t, -1, keepdims=True)
    d = t - mu
    var = jnp.mean(d * d, -1, keepdims=True)
    return d / jnp.sqrt(var + 1e-5) * g + b


def _wspec(shape):
    nd = len(shape)
    return pl.BlockSpec(shape, lambda i: (0,) * nd)


def _bspec(shape):
    return pl.BlockSpec(shape, lambda i: (i,) + (0,) * (len(shape) - 1))


_u32 = jnp.uint32
_u16 = jnp.uint16
_i32 = jnp.int32


def _pack_proj(hw):
    u = lax.bitcast_convert_type(hw.astype(_bf16), _u16)
    lo = u[:, 0:64].astype(_u32)
    hi = u[:, 64:128].astype(_u32)
    return lax.bitcast_convert_type(lo | (hi << 16), _i32)


def _unpack_proj(w):
    wu = lax.bitcast_convert_type(w, _u32)
    lo = lax.bitcast_convert_type((wu & 0xFFFF).astype(_u16), _bf16)
    hi = lax.bitcast_convert_type((wu >> 16).astype(_u16), _bf16)
    return jnp.concatenate([lo.astype(_f32), hi.astype(_f32)], axis=1)


def _tables(h, x16, wa, wb):
    xi = lax.bitcast_convert_type(x16, _i32)
    z = jnp.zeros((h.shape[0], TW - 80), _i32)
    pa = _pack_proj(jnp.dot(h, wa[...], preferred_element_type=_f32))
    pb = _pack_proj(jnp.dot(h, wb[...], preferred_element_type=_f32))
    a = jnp.concatenate([pa, xi, z], axis=1)
    b = jnp.concatenate([pb, xi, z], axis=1)
    return a, b


def _enc_body(hin, x16, new, neb, neg, nebeta, wa, wb, h_out, a_out, b_out):
    hp = _silu(jnp.dot(hin[...], new[...], preferred_element_type=_f32) + neb[...])
    h = _ln(hp, neg[...], nebeta[...])
    h_out[...] = h
    a_out[...], b_out[...] = _tables(h, x16[...], wa, wb)


def _edge_enc_body(eap, eew, eeb, out):
    out[...] = _silu(jnp.dot(eap[...], eew[...], preferred_element_type=_f32)
                     + eeb[...]).astype(_bf16)


def _edge_body(has_coord, av, bv, ea, wd, wc, eb1, ew2, eb2, cw1, cb1, cw2,
               m_out, t_out=None):
    a = av[...]
    b = bv[...]
    hsum = _unpack_proj(a[:, 0:64]) + _unpack_proj(b[:, 0:64])
    xd16 = (lax.bitcast_convert_type(a[:, 64:80], _f32)
            - lax.bitcast_convert_type(b[:, 64:80], _f32))
    dist_sq = jnp.sum(xd16 * xd16, axis=1, keepdims=True)
    e1 = (hsum
          + jnp.dot(ea[...].astype(_f32), wd[...], preferred_element_type=_f32)
          + dist_sq * wc[...] + eb1[...])
    m = _silu(jnp.dot(_silu(e1), ew2[...], preferred_element_type=_f32) + eb2[...])
    m_out[...] = m
    if has_coord:
        lane8 = lax.broadcasted_iota(jnp.int32, (EBLK, 8), 1)
        t = _silu(jnp.dot(m, cw1[...], preferred_element_type=_f32) + cb1[...])
        c = jnp.tanh(jnp.dot(t, cw2[...], preferred_element_type=_f32))
        dist = jnp.sqrt(dist_sq + 1e-8)
        tail8 = jnp.where(lane8 == 3, 1.0, xd16[:, 0:8] * (c / dist))
        t_out[...] = tail8.T


def _node_body(has_coord, *refs):
    if has_coord:
        (h_in, x16, p0, p1, t0, t1, nw1h, nw1m, nb1, nw2, nb2, lng, lnb,
         wa, wb, h_out, x_out, a_out, b_out) = refs
    else:
        h_in, p0, p1, nw1h, nw1m, nb1, nw2, nb2, lng, lnb, h_out = refs
    h = h_in[...]
    m_i = p0[:, 0:128] + p1[:, 0:128]
    hu = _silu(jnp.dot(h, nw1h[...], preferred_element_type=_f32)
               + jnp.dot(m_i, nw1m[...], preferred_element_type=_f32)
               + nb1[...])
    hu = jnp.dot(hu, nw2[...], preferred_element_type=_f32) + nb2[...]
    hn = _ln(h + hu, lng[...], lnb[...])
    h_out[...] = hn
    if has_coord:
        tail = t0[0] + t1[0]
        deg = jnp.maximum(tail[:, 3:4], 1.0)
        xu = tail[:, 0:3] / deg
        xn = x16[...] + jnp.concatenate(
            [xu, jnp.zeros((NBLK, 13), _f32)], axis=1)
        x_out[...] = xn
        a_out[...], b_out[...] = _tables(hn, xn, wa, wb)


def _pool_body(h_ref, bp_ref, pw1, pb1, pw2, pb2, cw1, cb1, cw2, cb2, cw3, cb3,
               out_ref):
    h = h_ref[...]
    bp = bp_ref[...]
    s = jnp.dot(jnp.tanh(jnp.dot(h, pw1[...], preferred_element_type=_f32)
                         + pb1[...]),
                pw2[...], preferred_element_type=_f32) + pb2[...]
    cols = lax.broadcasted_iota(jnp.int32, (NPAD, B), 1)
    m = bp == cols
    mf = m.astype(_f32)
    dn = (((0,), (0,)), ((), ()))
    smax = jnp.max(jnp.where(m, s, -1e30), axis=0, keepdims=True)
    sg = jnp.sum(jnp.where(m, smax, 0.0), axis=1, keepdims=True)
    sexp = jnp.exp(s - sg)
    ssum = lax.dot_general(sexp, mf, dn, preferred_element_type=_f32)
    sden = jnp.sum(jnp.where(m, ssum, 0.0), axis=1, keepdims=True)
    w = jnp.where(bp >= 0, sexp / (sden + 1e-16), 0.0)
    g = lax.dot_general(mf, h * w, dn, preferred_element_type=_f32)
    inv = 0.9999950000374996
    z = _silu(jnp.dot(g, cw1[...], preferred_element_type=_f32) + cb1[...]) * inv
    z = _silu(jnp.dot(z, cw2[...], preferred_element_type=_f32) + cb2[...]) * inv
    out_ref[...] = jnp.dot(z, cw3[...], preferred_element_type=_f32) + cb3[...]


def _gather_body(ta, tb, rows_hbm, cols_hbm, oa, ob, idxa, idxb,
                 a0, a1, b0, b1, sga0, sga1, sgb0, sgb1,
                 swa0, swa1, swb0, swb1):
    wid = lax.axis_index("c") * NSUB + lax.axis_index("s")
    cbase = wid * CPT
    ebase = cbase * CHUNK
    pltpu.sync_copy(rows_hbm.at[pl.ds(cbase, CPT)], idxa)
    pltpu.sync_copy(cols_hbm.at[pl.ds(cbase, CPT)], idxb)
    abufs = (a0, a1)
    bbufs = (b0, b1)
    sgas = (sga0, sga1)
    sgbs = (sgb0, sgb1)
    swas = (swa0, swa1)
    swbs = (swb0, swb1)

    pltpu.async_copy(ta.at[idxa.at[0]], a0, sga0)
    pltpu.async_copy(tb.at[idxb.at[0]], b0, sgb0)

    @pl.loop(0, CPT, step=2)
    def _pipe(t0):
        for par in range(2):
            t = t0 + par
            ab, bb = abufs[par], bbufs[par]
            sga, sgb = sgas[par], sgbs[par]
            swa, swb = swas[par], swbs[par]
            oab, obb = abufs[1 - par], bbufs[1 - par]
            osga, osgb = sgas[1 - par], sgbs[1 - par]
            oswa, oswb = swas[1 - par], swbs[1 - par]

            @pl.when(t >= 1)
            def _():
                off = ebase + (t - 1) * CHUNK
                pltpu.make_async_copy(oab, oa.at[pl.ds(off, CHUNK)],
                                      oswa).wait()
                pltpu.make_async_copy(obb, ob.at[pl.ds(off, CHUNK)],
                                      oswb).wait()

            @pl.when(t + 1 < CPT)
            def _():
                pltpu.async_copy(ta.at[idxa.at[t + 1]], oab, osga)
                pltpu.async_copy(tb.at[idxb.at[t + 1]], obb, osgb)

            off = ebase + t * CHUNK
            pltpu.make_async_copy(ta.at[idxa.at[t]], ab, sga).wait()
            pltpu.async_copy(ab, oa.at[pl.ds(off, CHUNK)], swa)
            pltpu.make_async_copy(tb.at[idxb.at[t]], bb, sgb).wait()
            pltpu.async_copy(bb, ob.at[pl.ds(off, CHUNK)], swb)

    offl = ebase + (CPT - 1) * CHUNK
    pltpu.make_async_copy(a1, oa.at[pl.ds(offl, CHUNK)], swa1).wait()
    pltpu.make_async_copy(b1, ob.at[pl.ds(offl, CHUNK)], swb1).wait()


_gather = pl.kernel(
    _gather_body,
    out_type=[
        jax.ShapeDtypeStruct((EPAD, TW), _i32),
        jax.ShapeDtypeStruct((EPAD, TW), _i32),
    ],
    mesh=plsc.VectorSubcoreMesh(core_axis_name="c", subcore_axis_name="s"),
    scratch_types=[
        pltpu.VMEM((CPT, CHUNK), jnp.int32),
        pltpu.VMEM((CPT, CHUNK), jnp.int32),
        pltpu.VMEM((CHUNK, TW), _i32),
        pltpu.VMEM((CHUNK, TW), _i32),
        pltpu.VMEM((CHUNK, TW), _i32),
        pltpu.VMEM((CHUNK, TW), _i32),
    ] + [pltpu.SemaphoreType.DMA] * 8,
)


def _scatter_body(idx_hbm, mv_hbm, zero_hbm, out_hbm, idx2d, buf0, buf1,
                  acc_sh, sv0, sv1, ss0, ss1):
    cid = lax.axis_index("c")
    sid = lax.axis_index("s")
    wid = cid * NSUB + sid
    rows = NPAD // NSUB
    rbase = sid * rows
    cbase = wid * CPT
    ebase = cbase * CHUNK
    pltpu.sync_copy(zero_hbm.at[pl.ds(rbase, rows)],
                    acc_sh.at[pl.ds(rbase, rows)])
    plsc.subcore_barrier()

    bufs = (buf0, buf1)
    svs = (sv0, sv1)
    sss = (ss0, ss1)

    pltpu.async_copy(mv_hbm.at[pl.ds(ebase, CHUNK)], buf0, sv0)

    @pl.loop(0, CPT, step=2)
    def _pipe(t0):
        for bpar in range(2):
            t = t0 + bpar
            buf, sv, ss = bufs[bpar], svs[bpar], sss[bpar]
            obuf, osv, oss = bufs[1 - bpar], svs[1 - bpar], sss[1 - bpar]

            @pl.when(t + 1 < CPT)
            def _():
                @pl.when(t >= 1)
                def _():
                    pltpu.make_async_copy(
                        obuf, acc_sh.at[idx2d.at[t - 1]], oss).wait()
                pltpu.async_copy(
                    mv_hbm.at[pl.ds(ebase + (t + 1) * CHUNK, CHUNK)],
                    obuf, osv)

            pltpu.make_async_copy(
                mv_hbm.at[pl.ds(ebase + t * CHUNK, CHUNK)], buf, sv).wait()
            pltpu.async_copy(buf, acc_sh.at[idx2d.at[t]], ss, add=True)

    pltpu.make_async_copy(buf0, acc_sh.at[idx2d.at[CPT - 2]], ss0).wait()
    pltpu.make_async_copy(buf1, acc_sh.at[idx2d.at[CPT - 1]], ss1).wait()
    plsc.subcore_barrier()
    obase = cid * NPAD + rbase
    pltpu.sync_copy(acc_sh.at[pl.ds(rbase, rows)],
                    out_hbm.at[pl.ds(obase, rows)])


_scatter = pl.kernel(
    _scatter_body,
    out_type=jax.ShapeDtypeStruct((NCORE * NPAD, SW), _f32),
    mesh=plsc.VectorSubcoreMesh(core_axis_name="c", subcore_axis_name="s"),
    scratch_types=[
        pltpu.VMEM((CPT, CHUNK), jnp.int32),
        pltpu.VMEM((CHUNK, SW), _f32),
        pltpu.VMEM((CHUNK, SW), _f32),
        pltpu.VMEM_SHARED((NPAD, SW), _f32),
        pltpu.SemaphoreType.DMA,
        pltpu.SemaphoreType.DMA,
        pltpu.SemaphoreType.DMA,
        pltpu.SemaphoreType.DMA,
    ],
)


def _scatter_mt_body(idx_hbm, mv_hbm, tv_hbm, zero_hbm, zflat_hbm,
                     out_m, out_t, ix0, ix1, v0, v1, t0b, t1b, i40, i41,
                     ztile, acc_m, acc_t,
                     si0, si1, sv0, sv1, ss0, ss1, st0, st1, se0, se1):
    cid = lax.axis_index("c")
    sid = lax.axis_index("s")
    wid = cid * NSUB + sid
    rows = NACC // NSUB
    rbase = sid * rows
    frows = 4 * NACC // NSUB
    fbase = sid * frows
    cbase = wid * CPT
    ebase = cbase * CHUNK
    pltpu.sync_copy(zero_hbm.at[pl.ds(rbase, rows)],
                    acc_m.at[pl.ds(rbase, rows)])
    pltpu.sync_copy(zflat_hbm.at[pl.ds(0, frows)], ztile)
    pltpu.sync_copy(ztile, acc_t.at[pl.ds(fbase, frows)])

    @pl.when(sid == 0)
    def _():
        pltpu.sync_copy(zero_hbm.at[pl.ds(0, NPAD - NACC)], v0)
        pltpu.sync_copy(v0, out_m.at[pl.ds(cid * NPAD + NACC, NPAD - NACC)])
        pltpu.sync_copy(ztile.at[pl.ds(0, 4 * (NPAD - NACC))],
                        out_t.at[pl.ds(cid * 4 * NPAD + 4 * NACC,
                                       4 * (NPAD - NACC))])
    plsc.subcore_barrier()

    vbufs = (v0, v1)
    tbufs = (t0b, t1b)
    ibufs = (i40, i41)
    ixbufs = (ix0, ix1)
    sis = (si0, si1)
    svs = (sv0, sv1)
    sss = (ss0, ss1)
    sts = (st0, st1)
    ses = (se0, se1)

    pltpu.async_copy(idx_hbm.at[cbase], ix0, si0)
    pltpu.async_copy(mv_hbm.at[pl.ds(ebase, CHUNK)], v0, sv0)
    pltpu.async_copy(tv_hbm.at[:, pl.ds(ebase, CHUNK)], t0b, st0)

    @pl.loop(0, CPT, step=2)
    def _pipe(tt0):
        for par in range(2):
            t = tt0 + par
            vb, tb, ib, ixb = (vbufs[par], tbufs[par], ibufs[par],
                               ixbufs[par])
            si, sv, ss, st, se = (sis[par], svs[par], sss[par], sts[par],
                                  ses[par])
            ovb, otb, oib, oixb = (vbufs[1 - par], tbufs[1 - par],
                                   ibufs[1 - par], ixbufs[1 - par])
            osi, osv, oss, ost, ose = (sis[1 - par], svs[1 - par],
                                       sss[1 - par], sts[1 - par],
                                       ses[1 - par])

            @pl.when(t + 1 < CPT)
            def _():
                @pl.when(t >= 1)
                def _():
                    pltpu.make_async_copy(
                        ovb, acc_m.at[oixb], oss).wait()
                    for d in range(4):
                        pltpu.make_async_copy(
                            otb.at[d], acc_t.at[oib.at[d]], ose).wait()
                off = ebase + (t + 1) * CHUNK
                pltpu.async_copy(idx_hbm.at[cbase + t + 1], oixb, osi)
                pltpu.async_copy(mv_hbm.at[pl.ds(off, CHUNK)], ovb, osv)
                pltpu.async_copy(tv_hbm.at[:, pl.ds(off, CHUNK)], otb, ost)

            pltpu.make_async_copy(idx_hbm.at[cbase + t], ixb, si).wait()
            pltpu.make_async_copy(
                mv_hbm.at[pl.ds(ebase + t * CHUNK, CHUNK)], vb, sv).wait()
            pltpu.async_copy(vb, acc_m.at[ixb], ss, add=True)

            pltpu.make_async_copy(
                tv_hbm.at[:, pl.ds(ebase + t * CHUNK, CHUNK)], tb, st).wait()
            for d in range(4):
                for g in range(CHUNK // 16):
                    sl = pl.ds(g * 16, 16)
                    ib[d, sl] = ixb[sl] * 4 + d
                pltpu.async_copy(tb.at[d], acc_t.at[ib.at[d]], se, add=True)

    for par in range(2):
        pltpu.make_async_copy(
            vbufs[par], acc_m.at[ixbufs[par]], sss[par]).wait()
        for d in range(4):
            pltpu.make_async_copy(
                tbufs[par].at[d], acc_t.at[ibufs[par].at[d]],
                ses[par]).wait()
    plsc.subcore_barrier()
    pltpu.sync_copy(acc_m.at[pl.ds(rbase, rows)],
                    out_m.at[pl.ds(cid * NPAD + rbase, rows)])
    pltpu.sync_copy(acc_t.at[pl.ds(fbase, frows)], ztile)
    pltpu.sync_copy(ztile, out_t.at[pl.ds(cid * 4 * NPAD + fbase, frows)])


_scatter_mt = pl.kernel(
    _scatter_mt_body,
    out_type=[
        jax.ShapeDtypeStruct((NCORE * NPAD, SW), _f32),
        jax.ShapeDtypeStruct((NCORE * 4 * NPAD,), _f32),
    ],
    mesh=plsc.VectorSubcoreMesh(core_axis_name="c", subcore_axis_name="s"),
    scratch_types=[
        pltpu.VMEM((CHUNK,), jnp.int32),
        pltpu.VMEM((CHUNK,), jnp.int32),
        pltpu.VMEM((CHUNK, SW), _f32),
        pltpu.VMEM((CHUNK, SW), _f32),
        pltpu.VMEM((8, CHUNK), _f32),
        pltpu.VMEM((8, CHUNK), _f32),
        pltpu.VMEM((8, CHUNK), jnp.int32),
        pltpu.VMEM((8, CHUNK), jnp.int32),
        pltpu.VMEM((4 * NACC // NSUB,), _f32),
        pltpu.VMEM_SHARED((NACC, SW), _f32),
        pltpu.VMEM_SHARED((4 * NACC,), _f32),
    ] + [pltpu.SemaphoreType.DMA] * 10,
)


def kernel(h, x, edge_index, edge_attr, batch, params):
    p = params
    r2 = lambda t: t.reshape(1, -1)

    hp = jnp.pad(h, ((0, NPAD - N), (0, 64 - 58)))
    x16 = jnp.pad(x, ((0, NPAD - N), (0, 13)))
    eap = jnp.pad(edge_attr, ((0, EPAD - E), (0, 4)))
    rowp = jnp.pad(edge_index[0], (0, EPAD - E),
                   constant_values=N).reshape(EPAD // CHUNK, CHUNK)
    colp = jnp.pad(edge_index[1], (0, EPAD - E),
                   constant_values=N).reshape(EPAD // CHUNK, CHUNK)
    bp = jnp.pad(batch, (0, NPAD - N), constant_values=-1).reshape(NPAD, 1)
    zeros_acc = jnp.zeros((NPAD, SW), _f32)
    zflat = zeros_acc.reshape(-1)
    new_p = jnp.pad(p['ne_w'], ((0, 6), (0, 0)))
    eew_p = jnp.pad(p['ee_w'], ((0, 4), (0, 0)))

    lw = p['layers']
    wa0 = lw[0]['ew1'][0:128]
    wb0 = lw[0]['ew1'][128:256]

    grid_n = (NPB,)
    grid_e = (EPAD // EBLK,)

    hcur, A, Bt = pl.pallas_call(
        _enc_body,
        grid=grid_n,
        in_specs=[
            _bspec((NBLK, 64)), _bspec((NBLK, 16)),
            _wspec((64, 128)), _wspec((1, 128)), _wspec((1, 128)),
            _wspec((1, 128)), _wspec((128, 128)), _wspec((128, 128)),
        ],
        out_specs=[_bspec((NBLK, 128)), _bspec((NBLK, TW)), _bspec((NBLK, TW))],
        out_shape=[
            jax.ShapeDtypeStruct((NPAD, 128), _f32),
            jax.ShapeDtypeStruct((NPAD, TW), _i32),
            jax.ShapeDtypeStruct((NPAD, TW), _i32),
        ],
    )(hp, x16, new_p, r2(p['ne_b']), r2(p['ne_g']), r2(p['ne_beta']), wa0, wb0)

    ea = pl.pallas_call(
        _edge_enc_body,
        grid=grid_e,
        in_specs=[_bspec((EBLK, 16)), _wspec((16, 128)), _wspec((1, 128))],
        out_specs=_bspec((EBLK, 128)),
        out_shape=jax.ShapeDtypeStruct((EPAD, 128), _bf16),
    )(eap, eew_p, r2(p['ee_b']))

    for i in range(L):
        lp = lw[i]
        has_coord = i < L - 1
        wc = lp['ew1'][256:257]
        wd = lp['ew1'][257:385]

        av, bv = _gather(A, Bt, rowp, colp)

        if has_coord:
            cw1, cb1, cw2 = lp['cw1'], r2(lp['cb1']), lp['cw2']
            n_out = 2
        else:
            cw1 = jnp.zeros((128, 128), _f32)
            cb1 = jnp.zeros((1, 128), _f32)
            cw2 = jnp.zeros((128, 1), _f32)
            n_out = 1
        eouts = pl.pallas_call(
            functools.partial(_edge_body, has_coord),
            grid=grid_e,
            in_specs=[
                _bspec((EBLK, TW)), _bspec((EBLK, TW)), _bspec((EBLK, 128)),
                _wspec((128, 128)), _wspec((1, 128)), _wspec((1, 128)),
                _wspec((128, 128)), _wspec((1, 128)),
                _wspec((128, 128)), _wspec((1, 128)), _wspec((128, 1)),
            ],
            out_specs=([_bspec((EBLK, SW)),
                        pl.BlockSpec((8, EBLK), lambda i: (0, i))][:n_out]),
            out_shape=([jax.ShapeDtypeStruct((EPAD, SW), _f32),
                        jax.ShapeDtypeStruct((8, EPAD), _f32)][:n_out]),
        )(av, bv, ea, wd, wc, r2(lp['eb1']), lp['ew2'], r2(lp['eb2']),
          cw1, cb1, cw2)
        if has_coord:
            mv, tv = eouts
        else:
            mv, = eouts

        if has_coord:
            parts_m, parts_t = _scatter_mt(rowp, mv, tv, zeros_acc, zflat)
            parts_t = parts_t.reshape(NCORE, NPAD, 4)
        else:
            parts_m = _scatter(rowp, mv, zeros_acc)

        nw1h = lp['nw1'][0:128]
        nw1m = lp['nw1'][128:256]
        if has_coord:
            wan = lw[i + 1]['ew1'][0:128]
            wbn = lw[i + 1]['ew1'][128:256]
            hcur, x16, A, Bt = pl.pallas_call(
                functools.partial(_node_body, True),
                grid=grid_n,
                in_specs=[
                    _bspec((NBLK, 128)), _bspec((NBLK, 16)),
                    _bspec((NBLK, SW)),
                    pl.BlockSpec((NBLK, SW), lambda j: (NPB + j, 0)),
                    pl.BlockSpec((1, NBLK, 4), lambda j: (0, j, 0)),
                    pl.BlockSpec((1, NBLK, 4), lambda j: (1, j, 0)),
                    _wspec((128, 128)), _wspec((128, 128)), _wspec((1, 128)),
                    _wspec((128, 128)), _wspec((1, 128)),
                    _wspec((1, 128)), _wspec((1, 128)),
                    _wspec((128, 128)), _wspec((128, 128)),
                ],
                out_specs=[_bspec((NBLK, 128)), _bspec((NBLK, 16)),
                           _bspec((NBLK, TW)), _bspec((NBLK, TW))],
                out_shape=[
                    jax.ShapeDtypeStruct((NPAD, 128), _f32),
                    jax.ShapeDtypeStruct((NPAD, 16), _f32),
                    jax.ShapeDtypeStruct((NPAD, TW), _i32),
                    jax.ShapeDtypeStruct((NPAD, TW), _i32),
                ],
            )(hcur, x16, parts_m, parts_m, parts_t, parts_t,
              nw1h, nw1m, r2(lp['nb1']), lp['nw2'],
              r2(lp['nb2']), r2(lp['ln_g']), r2(lp['ln_b']), wan, wbn)
        else:
            hcur = pl.pallas_call(
                functools.partial(_node_body, False),
                grid=grid_n,
                in_specs=[
                    _bspec((NBLK, 128)),
                    _bspec((NBLK, SW)),
                    pl.BlockSpec((NBLK, SW), lambda j: (NPB + j, 0)),
                    _wspec((128, 128)), _wspec((128, 128)), _wspec((1, 128)),
                    _wspec((128, 128)), _wspec((1, 128)),
                    _wspec((1, 128)), _wspec((1, 128)),
                ],
                out_specs=_bspec((NBLK, 128)),
                out_shape=jax.ShapeDtypeStruct((NPAD, 128), _f32),
            )(hcur, parts_m, parts_m, nw1h, nw1m, r2(lp['nb1']), lp['nw2'],
              r2(lp['nb2']), r2(lp['ln_g']), r2(lp['ln_b']))

    out = pl.pallas_call(
        _pool_body,
        out_shape=jax.ShapeDtypeStruct((B, 1), _f32),
    )(hcur, bp, p['pw1'], r2(p['pb1']), p['pw2'], r2(p['pb2']),
      p['cw1'], r2(p['cb1']), p['cw2'], r2(p['cb2']), p['cw3'], r2(p['cb3']))
    return out
